# baseline (speedup 1.0000x reference)
import functools
import math

import jax
import jax.numpy as jnp
from jax import lax
from jax.experimental import pallas as pl
from jax.experimental.pallas import tpu as pltpu
from jax.experimental.pallas import tpu_sc as plsc

B = 4096
IN_DIM = 768
SIG_DIM = 32
E = 8
EXPERT_DIM = 768
HID = E * 2

KA = IN_DIM + 128
TB = 512
NB = B // TB
G = NB + E - 1

NC = 2
NS = 16
NW = NC * NS
RPW = B // NW

_SQRT2 = math.sqrt(2.0)


def _router_kernel(x_ref, s2_ref, w1x_ref, w1s_ref, b1_ref, w2_ref, b2_ref,
                   xaug_ref, idx_ref):
    xb = x_ref[...]
    h = (jnp.dot(xb, w1x_ref[...], preferred_element_type=jnp.float32)
         + jnp.dot(s2_ref[...], w1s_ref[...], preferred_element_type=jnp.float32)
         + b1_ref[...])
    h = 0.5 * h * (1.0 + jax.lax.erf(h / _SQRT2))
    logits = jnp.dot(h, w2_ref[...], preferred_element_type=jnp.float32) + b2_ref[...]
    m = jnp.max(logits, axis=-1, keepdims=True)
    ssum = jnp.sum(jnp.exp(logits - m), axis=-1, keepdims=True)
    w = 1.0 / ssum
    idx = jnp.argmax(logits, axis=-1)[:, None]

    xaug_ref[:, :IN_DIM] = w * xb
    lane = lax.broadcasted_iota(jnp.int32, (TB, KA - IN_DIM), 1)
    xaug_ref[:, IN_DIM:] = jnp.where(lane == 0, w, 0.0)
    idx_ref[...] = idx


def _run_router(x, s2, w1x, w1s, b1, W2, b2):
    return pl.pallas_call(
        _router_kernel,
        grid=(NB,),
        in_specs=[
            pl.BlockSpec((TB, IN_DIM), lambda i: (i, 0)),
            pl.BlockSpec((TB, SIG_DIM + 3), lambda i: (i, 0)),
            pl.BlockSpec(w1x.shape, lambda i: (0, 0)),
            pl.BlockSpec(w1s.shape, lambda i: (0, 0)),
            pl.BlockSpec((1, HID), lambda i: (0, 0)),
            pl.BlockSpec(W2.shape, lambda i: (0, 0)),
            pl.BlockSpec((1, E), lambda i: (0, 0)),
        ],
        out_specs=[
            pl.BlockSpec((TB, KA), lambda i: (i, 0)),
            pl.BlockSpec((TB, 1), lambda i: (i, 0)),
        ],
        out_shape=[
            jax.ShapeDtypeStruct((B, KA), jnp.float32),
            jax.ShapeDtypeStruct((B, 1), jnp.int32),
        ],
    )(x, s2, w1x, w1s, b1, W2, b2)


def _permute_rows(src, idxs, d):
    mesh = plsc.VectorSubcoreMesh(core_axis_name="c", subcore_axis_name="s")

    @functools.partial(
        pl.kernel,
        out_type=jax.ShapeDtypeStruct((B, d), jnp.float32),
        mesh=mesh,
        scratch_types=[
            pltpu.VMEM((RPW,), jnp.int32),
            pltpu.VMEM((RPW, d), jnp.float32),
            pltpu.SemaphoreType.DMA,
        ],
    )
    def _perm(src_hbm, idx_hbm, out_hbm, idx_v, rows_v, sem):
        wid = lax.axis_index("s") * NC + lax.axis_index("c")
        base = wid * RPW
        pltpu.sync_copy(idx_hbm.at[pl.ds(base, RPW)], idx_v)
        pltpu.async_copy(src_hbm.at[idx_v], rows_v, sem).wait()
        pltpu.sync_copy(rows_v, out_hbm.at[pl.ds(base, RPW)])

    return _perm(src, idxs)


def _scatter_rows(src, idxs, d):
    mesh = plsc.VectorSubcoreMesh(core_axis_name="c", subcore_axis_name="s")

    @functools.partial(
        pl.kernel,
        out_type=jax.ShapeDtypeStruct((B, d), jnp.float32),
        mesh=mesh,
        scratch_types=[
            pltpu.VMEM((RPW,), jnp.int32),
            pltpu.VMEM((RPW, d), jnp.float32),
            pltpu.SemaphoreType.DMA,
        ],
    )
    def _scat(src_hbm, idx_hbm, out_hbm, idx_v, rows_v, sem):
        wid = lax.axis_index("s") * NC + lax.axis_index("c")
        base = wid * RPW
        pltpu.sync_copy(idx_hbm.at[pl.ds(base, RPW)], idx_v)
        pltpu.sync_copy(src_hbm.at[pl.ds(base, RPW)], rows_v)
        pltpu.async_copy(rows_v, out_hbm.at[idx_v], sem).wait()

    return _scat(src, idxs)


def _grouped_kernel(blk_ref, we_ref_idx, first_ref, off_ref, es_ref,
                    xs_ref, we_ref, be_ref, out_ref):
    g = pl.program_id(0)
    b = blk_ref[g]
    e = es_ref[g]
    lo = off_ref[e]
    hi = off_ref[e + 1]
    rowpos = b * TB + lax.broadcasted_iota(jnp.int32, (TB, 1), 0)
    mask = ((rowpos >= lo) & (rowpos < hi)).astype(jnp.float32)
    xs = xs_ref[...]
    x768 = xs[:, :IN_DIM]
    wcol = xs[:, IN_DIM:IN_DIM + 1]
    prod = jnp.dot(x768, we_ref[0], preferred_element_type=jnp.float32)
    contrib = mask * (prod + wcol * be_ref[0])

    @pl.when(first_ref[g] == 1)
    def _():
        out_ref[...] = contrib

    @pl.when(first_ref[g] == 0)
    def _():
        out_ref[...] = out_ref[...] + contrib


def _run_grouped(blk, wi, first, off_ext, es, xsorted, We, be3):
    grid_spec = pltpu.PrefetchScalarGridSpec(
        num_scalar_prefetch=5,
        grid=(G,),
        in_specs=[
            pl.BlockSpec((TB, KA), lambda g, blk, wi, fi, off, es: (blk[g], 0)),
            pl.BlockSpec((1, IN_DIM, EXPERT_DIM),
                         lambda g, blk, wi, fi, off, es: (wi[g], 0, 0)),
            pl.BlockSpec((1, 1, EXPERT_DIM),
                         lambda g, blk, wi, fi, off, es: (wi[g], 0, 0)),
        ],
        out_specs=pl.BlockSpec((TB, EXPERT_DIM),
                               lambda g, blk, wi, fi, off, es: (blk[g], 0)),
    )
    return pl.pallas_call(
        _grouped_kernel,
        grid_spec=grid_spec,
        out_shape=jax.ShapeDtypeStruct((B, EXPERT_DIM), jnp.float32),
    )(blk, wi, first, off_ext, es, xsorted, We, be3)


@jax.jit
def kernel(x, physical_signature, task_context, resource_state,
           W1, b1, W2, b2, We, be):
    s2 = jnp.concatenate([physical_signature, task_context, resource_state],
                         axis=-1)
    w1x = W1[:IN_DIM]
    w1s = W1[IN_DIM:]

    xaug, idxo = _run_router(x, s2, w1x, w1s, b1[None, :], W2, b2[None, :])
    idx = idxo[:, 0]

    i32 = jnp.int32
    ohb = idx[:, None] == jnp.arange(E, dtype=i32)[None, :]
    oh = ohb.astype(i32)
    c = jnp.cumsum(oh, axis=0)
    counts = c[-1]
    ends = jnp.cumsum(counts)
    off = jnp.concatenate([jnp.zeros((1,), i32), ends]).astype(i32)
    off_ext = jnp.concatenate([off, jnp.full((1,), B, i32)])
    rank = jnp.sum(jnp.where(ohb, c, 0), axis=1) - 1
    start_of = jnp.sum(jnp.where(ohb, off[None, :E], 0), axis=1)
    position = start_of + rank

    bb = jnp.arange(NB, dtype=i32)
    e_lo = jnp.sum((ends[None, :] <= bb[:, None] * TB).astype(i32), axis=1)
    e_hi = jnp.sum((ends[None, :] <= (bb[:, None] + 1) * TB - 1).astype(i32),
                   axis=1)
    spans = e_hi - e_lo + 1
    start = jnp.concatenate([jnp.zeros((1,), i32),
                             jnp.cumsum(spans)]).astype(i32)
    g = jnp.arange(G, dtype=i32)
    b_of_g = jnp.clip(
        jnp.sum((start[None, 1:] <= g[:, None]).astype(i32), axis=1),
        0, NB - 1)
    gsb = jnp.sum(jnp.where(bb[None, :] == b_of_g[:, None],
                            start[None, :NB], 0), axis=1)
    gel = jnp.sum(jnp.where(bb[None, :] == b_of_g[:, None],
                            e_lo[None, :], 0), axis=1)
    e_of_g = jnp.clip(gel + (g - gsb), 0, E).astype(i32)
    first_of_g = (g == gsb).astype(i32)
    we_of_g = jnp.minimum(e_of_g, E - 1)

    xsorted = _scatter_rows(xaug, position, KA)
    be3 = be[:, None, :]
    sortedraw = _run_grouped(b_of_g, we_of_g, first_of_g, off_ext, e_of_g,
                             xsorted, We, be3)
    out = _permute_rows(sortedraw, position, EXPERT_DIM)
    return out


# speedup vs baseline: 1.4058x; 1.4058x over previous
import math

import jax
import jax.numpy as jnp
from jax import lax
from jax.experimental import pallas as pl
from jax.experimental.pallas import tpu as pltpu

B = 4096
IN_DIM = 768
SIG_DIM = 32
E = 8
EXPERT_DIM = 768
HID = E * 2

_SQRT2 = math.sqrt(2.0)


def _moe_kernel(x_ref, s2_ref, w1x_ref, w1s_ref, b1_ref, w2_ref, b2_ref,
                we_ref, be_ref, out_ref, x16_s, w_s, idx_s):
    j = pl.program_id(0)

    @pl.when(j == 0)
    def _():
        xb = x_ref[...]
        h = (jnp.dot(xb, w1x_ref[...], preferred_element_type=jnp.float32)
             + jnp.dot(s2_ref[...], w1s_ref[...],
                       preferred_element_type=jnp.float32)
             + b1_ref[...])
        h = 0.5 * h * (1.0 + jax.lax.erf(h / _SQRT2))
        logits = (jnp.dot(h, w2_ref[...], preferred_element_type=jnp.float32)
                  + b2_ref[...])
        m = jnp.max(logits, axis=-1, keepdims=True)
        ssum = jnp.sum(jnp.exp(logits - m), axis=-1, keepdims=True)
        w_s[...] = 1.0 / ssum
        idx_s[...] = jnp.argmax(logits, axis=-1)[:, None]
        x16_s[...] = xb.astype(jnp.bfloat16)
        out_ref[...] = jnp.zeros((B, EXPERT_DIM), jnp.float32)

    @pl.when(j > 0)
    def _():
        wj = jnp.where(idx_s[...] == j - 1, w_s[...], 0.0)
        we16 = we_ref[0].astype(jnp.bfloat16)
        ex = lax.dot_general(x16_s[...], we16, (((1,), (0,)), ((), ())),
                             precision=lax.Precision.DEFAULT,
                             preferred_element_type=jnp.float32)
        out_ref[...] = out_ref[...] + wj * (ex + be_ref[0])


@jax.jit
def kernel(x, physical_signature, task_context, resource_state,
           W1, b1, W2, b2, We, be):
    s2 = jnp.concatenate([physical_signature, task_context, resource_state],
                         axis=-1)
    w1x = W1[:IN_DIM]
    w1s = W1[IN_DIM:]

    out = pl.pallas_call(
        _moe_kernel,
        grid=(E + 1,),
        in_specs=[
            pl.BlockSpec((B, IN_DIM), lambda j: (0, 0)),
            pl.BlockSpec((B, SIG_DIM + 3), lambda j: (0, 0)),
            pl.BlockSpec(w1x.shape, lambda j: (0, 0)),
            pl.BlockSpec(w1s.shape, lambda j: (0, 0)),
            pl.BlockSpec((1, HID), lambda j: (0, 0)),
            pl.BlockSpec(W2.shape, lambda j: (0, 0)),
            pl.BlockSpec((1, E), lambda j: (0, 0)),
            pl.BlockSpec((1, IN_DIM, EXPERT_DIM),
                         lambda j: (jnp.maximum(j - 1, 0), 0, 0)),
            pl.BlockSpec((1, 1, EXPERT_DIM),
                         lambda j: (jnp.maximum(j - 1, 0), 0, 0)),
        ],
        out_specs=pl.BlockSpec((B, EXPERT_DIM), lambda j: (0, 0)),
        out_shape=jax.ShapeDtypeStruct((B, EXPERT_DIM), jnp.float32),
        scratch_shapes=[
            pltpu.VMEM((B, IN_DIM), jnp.bfloat16),
            pltpu.VMEM((B, 1), jnp.float32),
            pltpu.VMEM((B, 1), jnp.int32),
        ],
    )(x, s2, w1x, w1s, b1[None, :], W2, b2[None, :], We, be[:, None, :])
    return out


# speedup vs baseline: 1.5162x; 1.0785x over previous
import math

import jax
import jax.numpy as jnp
from jax.experimental import pallas as pl

B = 4096
IN_DIM = 768
SIG_DIM = 32
E = 8
EXPERT_DIM = 768
HID = E * 2

TB = 1024
NB = B // TB

_SQRT2 = math.sqrt(2.0)


def _moe_kernel(x_ref, s2_ref, w1x_ref, w1s_ref, b1_ref, w2_ref, b2_ref,
                we_ref, be_ref, out_ref):
    xb = x_ref[...]
    h = (jnp.dot(xb, w1x_ref[...], preferred_element_type=jnp.float32)
         + jnp.dot(s2_ref[...], w1s_ref[...], preferred_element_type=jnp.float32)
         + b1_ref[...])
    h = 0.5 * h * (1.0 + jax.lax.erf(h / _SQRT2))
    logits = jnp.dot(h, w2_ref[...], preferred_element_type=jnp.float32) + b2_ref[...]
    m = jnp.max(logits, axis=-1, keepdims=True)
    ssum = jnp.sum(jnp.exp(logits - m), axis=-1, keepdims=True)
    w = 1.0 / ssum
    idx = jnp.argmax(logits, axis=-1)[:, None]

    acc = jnp.zeros((TB, EXPERT_DIM), dtype=jnp.float32)
    for j in range(E):
        wj = jnp.where(idx == j, w, 0.0)
        ex = jnp.dot(xb, we_ref[j], preferred_element_type=jnp.float32) + be_ref[j]
        acc = acc + wj * ex
    out_ref[...] = acc


@jax.jit
def kernel(x, physical_signature, task_context, resource_state,
           W1, b1, W2, b2, We, be):
    s2 = jnp.concatenate([physical_signature, task_context, resource_state],
                         axis=-1)
    w1x = W1[:IN_DIM]
    w1s = W1[IN_DIM:]

    grid = (NB,)
    out = pl.pallas_call(
        _moe_kernel,
        grid=grid,
        in_specs=[
            pl.BlockSpec((TB, IN_DIM), lambda i: (i, 0)),
            pl.BlockSpec((TB, SIG_DIM + 3), lambda i: (i, 0)),
            pl.BlockSpec(w1x.shape, lambda i: (0, 0)),
            pl.BlockSpec(w1s.shape, lambda i: (0, 0)),
            pl.BlockSpec((1, HID), lambda i: (0, 0)),
            pl.BlockSpec(W2.shape, lambda i: (0, 0)),
            pl.BlockSpec((1, E), lambda i: (0, 0)),
            pl.BlockSpec(We.shape, lambda i: (0, 0, 0)),
            pl.BlockSpec(be.shape, lambda i: (0, 0)),
        ],
        out_specs=pl.BlockSpec((TB, EXPERT_DIM), lambda i: (i, 0)),
        out_shape=jax.ShapeDtypeStruct((B, EXPERT_DIM), jnp.float32),
    )(x, s2, w1x, w1s, b1[None, :], W2, b2[None, :], We, be)
    return out
